# baseline (speedup 1.0000x reference)
import functools

import jax
import jax.numpy as jnp
from jax.experimental import pallas as pl
from jax.experimental.pallas import tpu as pltpu


def _pool_body(x_ref, o_ref, *, c, lane_groups, rem, inv_l):
    x = x_ref[...].astype(jnp.float32)
    if lane_groups >= 1:
        m = x[:, :, 0:128]
        s = m
        for k in range(1, lane_groups):
            part = x[:, :, 128 * k:128 * (k + 1)]
            m = jnp.maximum(m, part)
            s = s + part
        mx = jnp.max(m, axis=-1)
        sm = jnp.sum(s, axis=-1)
        if rem:
            tail = x[:, :, 128 * lane_groups:]
            mx = jnp.maximum(mx, jnp.max(tail, axis=-1))
            sm = sm + jnp.sum(tail, axis=-1)
    else:
        mx = jnp.max(x, axis=-1)
        sm = jnp.sum(x, axis=-1)
    o_ref[:, :c] = mx.astype(o_ref.dtype)
    o_ref[:, c:] = (sm * inv_l).astype(o_ref.dtype)


def kernel(x):
    N, C, L = x.shape
    tb = 16
    while tb > 1 and N % tb != 0:
        tb //= 2
    lane_groups = L // 128
    rem = L % 128

    out = pl.pallas_call(
        functools.partial(_pool_body, c=C, lane_groups=lane_groups, rem=rem,
                          inv_l=float(1.0 / L)),
        out_shape=jax.ShapeDtypeStruct((N, 2 * C), x.dtype),
        grid=(N // tb,),
        in_specs=[pl.BlockSpec((tb, C, L), lambda i: (i, 0, 0))],
        out_specs=pl.BlockSpec((tb, 2 * C), lambda i: (i, 0)),
        compiler_params=pltpu.CompilerParams(
            dimension_semantics=("parallel",),
            vmem_limit_bytes=100 << 20,
        ),
        cost_estimate=pl.CostEstimate(
            flops=2 * N * C * L,
            transcendentals=0,
            bytes_accessed=N * C * L * x.dtype.itemsize,
        ),
    )(x)
    return out.reshape(N, 2 * C, 1)


# speedup vs baseline: 1.0183x; 1.0183x over previous
import functools

import jax
import jax.numpy as jnp
from jax.experimental import pallas as pl
from jax.experimental.pallas import tpu as pltpu


def _pool_one(x, o_ref, rows, *, c, lane_groups, rem, inv_l):
    if lane_groups >= 1:
        m = x[:, :, 0:128]
        s = m
        for k in range(1, lane_groups):
            part = x[:, :, 128 * k:128 * (k + 1)]
            m = jnp.maximum(m, part)
            s = s + part
        mx = jnp.max(m, axis=-1)
        sm = jnp.sum(s, axis=-1)
        if rem:
            tail = x[:, :, 128 * lane_groups:]
            mx = jnp.maximum(mx, jnp.max(tail, axis=-1))
            sm = sm + jnp.sum(tail, axis=-1)
    else:
        mx = jnp.max(x, axis=-1)
        sm = jnp.sum(x, axis=-1)
    o_ref[rows, :c] = mx.astype(o_ref.dtype)
    o_ref[rows, c:] = (sm * inv_l).astype(o_ref.dtype)


def _pool_body(*refs, c, tb, lane_groups, rem, inv_l):
    x_refs = refs[:-1]
    o_ref = refs[-1]
    for j, x_ref in enumerate(x_refs):
        x = x_ref[...].astype(jnp.float32)
        _pool_one(x, o_ref, pl.ds(j * tb, tb), c=c, lane_groups=lane_groups,
                  rem=rem, inv_l=inv_l)


def kernel(x):
    N, C, L = x.shape
    lane_groups = L // 128
    rem = L % 128
    cost = pl.CostEstimate(
        flops=2 * N * C * L,
        transcendentals=0,
        bytes_accessed=N * C * L * x.dtype.itemsize + N * 2 * C * x.dtype.itemsize,
    )

    tb, streams = 8, 4
    if N % (streams * tb) != 0 or N // (streams * tb) < 2:
        streams = 1
        tb = max(1, min(N, 16))
        while N % tb != 0:
            tb -= 1

    in_specs = [
        pl.BlockSpec((tb, C, L),
                     functools.partial(lambda j, i: (streams * i + j, 0, 0), j))
        for j in range(streams)
    ]
    out = pl.pallas_call(
        functools.partial(_pool_body, c=C, tb=tb, lane_groups=lane_groups,
                          rem=rem, inv_l=float(1.0 / L)),
        out_shape=jax.ShapeDtypeStruct((N, 2 * C), x.dtype),
        grid=(N // (streams * tb),),
        in_specs=in_specs,
        out_specs=pl.BlockSpec((streams * tb, 2 * C), lambda i: (i, 0)),
        compiler_params=pltpu.CompilerParams(
            dimension_semantics=("parallel",),
            vmem_limit_bytes=48 << 20,
        ),
        cost_estimate=cost,
    )(*([x] * streams))
    return out.reshape(N, 2 * C, 1)


# speedup vs baseline: 1.0206x; 1.0023x over previous
import functools

import jax
import jax.numpy as jnp
from jax.experimental import pallas as pl
from jax.experimental.pallas import tpu as pltpu


def _pool_one(x, o_ref, rows, *, c, lane_groups, rem, inv_l):
    if lane_groups >= 1:
        m = x[:, :, 0:128]
        s = m
        for k in range(1, lane_groups):
            part = x[:, :, 128 * k:128 * (k + 1)]
            m = jnp.maximum(m, part)
            s = s + part
        mx = jnp.max(m, axis=-1)
        sm = jnp.sum(s, axis=-1)
        if rem:
            tail = x[:, :, 128 * lane_groups:]
            mx = jnp.maximum(mx, jnp.max(tail, axis=-1))
            sm = sm + jnp.sum(tail, axis=-1)
    else:
        mx = jnp.max(x, axis=-1)
        sm = jnp.sum(x, axis=-1)
    o_ref[rows, :c] = mx.astype(o_ref.dtype)
    o_ref[rows, c:] = (sm * inv_l).astype(o_ref.dtype)


def _pool_body(*refs, c, tb, lane_groups, rem, inv_l):
    x_refs = refs[:-1]
    o_ref = refs[-1]
    for j, x_ref in enumerate(x_refs):
        x = x_ref[...].astype(jnp.float32)
        _pool_one(x, o_ref, pl.ds(j * tb, tb), c=c, lane_groups=lane_groups,
                  rem=rem, inv_l=inv_l)


def kernel(x):
    N, C, L = x.shape
    lane_groups = L // 128
    rem = L % 128
    cost = pl.CostEstimate(
        flops=2 * N * C * L,
        transcendentals=0,
        bytes_accessed=N * C * L * x.dtype.itemsize + N * 2 * C * x.dtype.itemsize,
    )

    tb, streams = 8, 2
    if N % (streams * tb) != 0 or N // (streams * tb) < 2:
        streams = 1
        tb = max(1, min(N, 16))
        while N % tb != 0:
            tb -= 1

    in_specs = [
        pl.BlockSpec((tb, C, L),
                     functools.partial(lambda j, i: (streams * i + j, 0, 0), j))
        for j in range(streams)
    ]
    out = pl.pallas_call(
        functools.partial(_pool_body, c=C, tb=tb, lane_groups=lane_groups,
                          rem=rem, inv_l=float(1.0 / L)),
        out_shape=jax.ShapeDtypeStruct((N, 2 * C), x.dtype),
        grid=(N // (streams * tb),),
        in_specs=in_specs,
        out_specs=pl.BlockSpec((streams * tb, 2 * C), lambda i: (i, 0)),
        compiler_params=pltpu.CompilerParams(
            dimension_semantics=("parallel",),
            vmem_limit_bytes=48 << 20,
        ),
        cost_estimate=cost,
    )(*([x] * streams))
    return out.reshape(N, 2 * C, 1)
